# baseline (speedup 1.0000x reference)
import functools
import math

import jax
import jax.numpy as jnp
from jax import lax
from jax.experimental import pallas as pl
from jax.experimental.pallas import tpu as pltpu
from jax.experimental.pallas import tpu_sc as plsc

NC = 2
NS = 16
LANES = 16
NW = NC * NS
KC = 80
F0 = 0.45
WROWS = 128


def _sc_mesh():
    return plsc.VectorSubcoreMesh(
        core_axis_name="c", subcore_axis_name="s", num_cores=NC, num_subcores=NS
    )


@functools.lru_cache(maxsize=None)
def _make_deg_kernel(n_pad, per_w):
    npt = n_pad // NS

    @functools.partial(
        pl.kernel,
        mesh=_sc_mesh(),
        out_type=jax.ShapeDtypeStruct((NC * n_pad,), jnp.float32),
        scratch_types=[
            pltpu.VMEM((per_w,), jnp.int32),
            pltpu.VMEM((n_pad,), jnp.float32),
            pltpu.VMEM((npt,), jnp.float32),
            pltpu.VMEM((npt,), jnp.float32),
            pltpu.VMEM_SHARED((NS * n_pad,), jnp.float32),
        ],
        compiler_params=pltpu.CompilerParams(needs_layout_passes=False),
    )
    def deg_kernel(cols_hbm, deg_hbm, colv, hist, acc, tbuf, hist_sh):
        cid = lax.axis_index("c")
        sid = lax.axis_index("s")
        wid = sid * NC + cid
        zero16 = jnp.zeros((LANES,), jnp.float32)
        one16 = jnp.ones((LANES,), jnp.float32)

        @pl.loop(0, n_pad // LANES)
        def _(r):
            hist[pl.ds(r * LANES, LANES)] = zero16

        pltpu.sync_copy(cols_hbm.at[pl.ds(wid * per_w, per_w)], colv)

        @pl.loop(0, per_w // LANES)
        def _(i):
            idx = colv[pl.ds(i * LANES, LANES)]
            plsc.addupdate_scatter(hist, [idx], one16)

        pltpu.sync_copy(hist, hist_sh.at[pl.ds(sid * n_pad, n_pad)])
        plsc.subcore_barrier()

        base = pl.multiple_of(sid * npt, npt)
        pltpu.sync_copy(hist_sh.at[pl.ds(base, npt)], acc)

        @pl.loop(1, NS)
        def _(t):
            pltpu.sync_copy(hist_sh.at[pl.ds(t * n_pad + base, npt)], tbuf)

            @pl.loop(0, npt // LANES)
            def _(j):
                sl = pl.ds(j * LANES, LANES)
                acc[sl] = acc[sl] + tbuf[sl]

        pltpu.sync_copy(acc, deg_hbm.at[pl.ds(cid * n_pad + base, npt)])

    return deg_kernel


@functools.lru_cache(maxsize=None)
def _make_scat_kernel(n_pad, d, per_w0, per_w1):
    npt = n_pad // NS
    wrows = 64
    nrounds = npt // wrows
    pw_max = max(per_w0, per_w1)
    nbuf = 2

    @functools.partial(
        pl.kernel,
        mesh=_sc_mesh(),
        out_type=jax.ShapeDtypeStruct((NC, n_pad, d), jnp.float32),
        scratch_types=[
            pltpu.VMEM((pw_max,), jnp.int32),
            pltpu.VMEM((pw_max,), jnp.int32),
            pltpu.VMEM((KC, d), jnp.float32),
            pltpu.VMEM((KC, d), jnp.float32),
            pltpu.VMEM_SHARED((n_pad, d), jnp.float32),
            pltpu.SemaphoreType.DMA,
            pltpu.SemaphoreType.DMA,
        ],
    )
    def scat_kernel(rows_hbm, cols_hbm, y_hbm, s_hbm, rowv, colv, gbuf0,
                    gbuf1, s_sp, sem0, sem1):
        cid = lax.axis_index("c")
        sid = lax.axis_index("s")
        gbufs = (gbuf0, gbuf1)
        sems = (sem0, sem1)
        zero16 = jnp.zeros((LANES,), jnp.float32)

        @pl.loop(0, wrows)
        def _(r):
            @pl.loop(0, d // LANES)
            def _(cc):
                gbuf0[r, pl.ds(cc * LANES, LANES)] = zero16

        @pl.loop(0, nrounds)
        def _(k):
            off = pl.multiple_of(sid * npt + k * wrows, wrows)
            pltpu.sync_copy(gbuf0.at[pl.ds(0, wrows)], s_sp.at[pl.ds(off, wrows)])

        plsc.subcore_barrier()

        def pipeline(base, per_w_c):
            nchunk = per_w_c // KC
            pltpu.sync_copy(rows_hbm.at[pl.ds(base, per_w_c)],
                            rowv.at[pl.ds(0, per_w_c)])
            pltpu.sync_copy(cols_hbm.at[pl.ds(base, per_w_c)],
                            colv.at[pl.ds(0, per_w_c)])
            for b in range(min(nbuf, nchunk)):
                eoff = pl.multiple_of(b * KC, 16)
                pltpu.async_copy(y_hbm.at[rowv.at[pl.ds(eoff, KC)]],
                                 gbufs[b], sems[b])

            @pl.loop(0, -(-nchunk // nbuf))
            def _(g):
                for b in range(nbuf):
                    ch = g * nbuf + b

                    @pl.when(ch < nchunk)
                    def _():
                        eoff = pl.multiple_of(ch * KC, 16)
                        pltpu.make_async_copy(
                            y_hbm.at[rowv.at[pl.ds(eoff, KC)]], gbufs[b],
                            sems[b]).wait()
                        pltpu.sync_copy(gbufs[b],
                                        s_sp.at[colv.at[pl.ds(eoff, KC)]],
                                        add=True)
                        nch = ch + nbuf

                        @pl.when(nch < nchunk)
                        def _():
                            noff = pl.multiple_of(nch * KC, 16)
                            pltpu.async_copy(
                                y_hbm.at[rowv.at[pl.ds(noff, KC)]],
                                gbufs[b], sems[b])

        @pl.when(cid == 0)
        def _():
            pipeline(pl.multiple_of(sid * per_w0, 16), per_w0)

        @pl.when(cid == 1)
        def _():
            pipeline(pl.multiple_of(NS * per_w0 + sid * per_w1, 16), per_w1)

        plsc.subcore_barrier()

        @pl.loop(0, nrounds)
        def _(k):
            off = pl.multiple_of(sid * npt + k * wrows, wrows)
            pltpu.sync_copy(s_sp.at[pl.ds(off, wrows)], gbuf0.at[pl.ds(0, wrows)])
            pltpu.sync_copy(gbuf0.at[pl.ds(0, wrows)],
                            s_hbm.at[cid, pl.ds(off, wrows)])

    return scat_kernel


def _lin_body(h_ref, w_ref, dis_ref, y_ref):
    x = jnp.dot(h_ref[...], w_ref[...], preferred_element_type=jnp.float32)
    y_ref[...] = x * dis_ref[...]


def _epi_body(s_ref, y_ref, dis_ref, b_ref, a_ref, lnw_ref, lnb_ref, out_ref):
    s = s_ref[0] + s_ref[1] + y_ref[...]
    pre = s * dis_ref[...] + b_ref[...]
    a = a_ref[0, 0]
    pre = jnp.where(pre >= 0, pre, a * pre)
    mean = jnp.mean(pre, axis=-1, keepdims=True)
    cent = pre - mean
    var = jnp.mean(cent * cent, axis=-1, keepdims=True)
    out_ref[...] = cent * lax.rsqrt(var + 1e-5) * lnw_ref[...] + lnb_ref[...]


def kernel(h_in, edge_index, W, b, prelu_a, ln_w, ln_b):
    n, d_in = h_in.shape
    d_out = W.shape[1]
    e = edge_index.shape[1]

    align = math.lcm(NW * LANES, NS * KC)
    ew = -(-e // align) * align
    n_pad = -(-n // (NS * WROWS)) * (NS * WROWS)

    rows = jnp.concatenate(
        [edge_index[0], jnp.zeros((ew - e,), jnp.int32)])
    cols = jnp.concatenate(
        [edge_index[1], jnp.full((ew - e,), n, jnp.int32)])

    pt = ew // NS
    per_w0 = min(max(round(F0 * pt / KC) * KC, KC), pt - KC)
    per_w1 = pt - per_w0

    deg_flat = _make_deg_kernel(n_pad, ew // NW)(cols)
    degt = deg_flat.reshape(NC, n_pad).sum(0)[:n] + 1.0
    dis = lax.rsqrt(degt)[:, None]

    br = 2000 if n % 2000 == 0 else 1000 if n % 1000 == 0 else 8
    grid = (n // br,)
    y = pl.pallas_call(
        _lin_body,
        grid=grid,
        in_specs=[
            pl.BlockSpec((br, d_in), lambda i: (i, 0)),
            pl.BlockSpec((d_in, d_out), lambda i: (0, 0)),
            pl.BlockSpec((br, 1), lambda i: (i, 0)),
        ],
        out_specs=pl.BlockSpec((br, d_out), lambda i: (i, 0)),
        out_shape=jax.ShapeDtypeStruct((n, d_out), jnp.float32),
    )(h_in, W, dis)

    s_parts = _make_scat_kernel(n_pad, d_out, per_w0, per_w1)(rows, cols, y)

    out = pl.pallas_call(
        _epi_body,
        grid=grid,
        in_specs=[
            pl.BlockSpec((NC, br, d_out), lambda i: (0, i, 0)),
            pl.BlockSpec((br, d_out), lambda i: (i, 0)),
            pl.BlockSpec((br, 1), lambda i: (i, 0)),
            pl.BlockSpec((1, d_out), lambda i: (0, 0)),
            pl.BlockSpec(memory_space=pltpu.SMEM),
            pl.BlockSpec((1, d_out), lambda i: (0, 0)),
            pl.BlockSpec((1, d_out), lambda i: (0, 0)),
        ],
        out_specs=pl.BlockSpec((br, d_out), lambda i: (i, 0)),
        out_shape=jax.ShapeDtypeStruct((n, d_out), jnp.float32),
    )(s_parts, y, dis, b.reshape(1, -1), prelu_a.reshape(1, 1),
      ln_w.reshape(1, -1), ln_b.reshape(1, -1))
    return out


# speedup vs baseline: 1.0534x; 1.0534x over previous
import functools
import math

import jax
import jax.numpy as jnp
from jax import lax
from jax.experimental import pallas as pl
from jax.experimental.pallas import tpu as pltpu
from jax.experimental.pallas import tpu_sc as plsc

NC = 2
NS = 16
LANES = 16
NW = NC * NS
KC = 80
F0 = 0.50
WROWS = 128


def _sc_mesh():
    return plsc.VectorSubcoreMesh(
        core_axis_name="c", subcore_axis_name="s", num_cores=NC, num_subcores=NS
    )


@functools.lru_cache(maxsize=None)
def _make_deg_kernel(n_pad, per_w):
    npt = n_pad // NS

    @functools.partial(
        pl.kernel,
        mesh=_sc_mesh(),
        out_type=jax.ShapeDtypeStruct((NC * n_pad,), jnp.float32),
        scratch_types=[
            pltpu.VMEM((per_w,), jnp.int32),
            pltpu.VMEM((n_pad,), jnp.float32),
            pltpu.VMEM((npt,), jnp.float32),
            pltpu.VMEM((npt,), jnp.float32),
            pltpu.VMEM_SHARED((NS * n_pad,), jnp.float32),
        ],
        compiler_params=pltpu.CompilerParams(needs_layout_passes=False),
    )
    def deg_kernel(cols_hbm, deg_hbm, colv, hist, acc, tbuf, hist_sh):
        cid = lax.axis_index("c")
        sid = lax.axis_index("s")
        wid = sid * NC + cid
        zero16 = jnp.zeros((LANES,), jnp.float32)
        one16 = jnp.ones((LANES,), jnp.float32)

        @pl.loop(0, n_pad // LANES)
        def _(r):
            hist[pl.ds(r * LANES, LANES)] = zero16

        pltpu.sync_copy(cols_hbm.at[pl.ds(wid * per_w, per_w)], colv)

        @pl.loop(0, per_w // LANES)
        def _(i):
            idx = colv[pl.ds(i * LANES, LANES)]
            plsc.addupdate_scatter(hist, [idx], one16)

        pltpu.sync_copy(hist, hist_sh.at[pl.ds(sid * n_pad, n_pad)])
        plsc.subcore_barrier()

        base = pl.multiple_of(sid * npt, npt)
        pltpu.sync_copy(hist_sh.at[pl.ds(base, npt)], acc)

        @pl.loop(1, NS)
        def _(t):
            pltpu.sync_copy(hist_sh.at[pl.ds(t * n_pad + base, npt)], tbuf)

            @pl.loop(0, npt // LANES)
            def _(j):
                sl = pl.ds(j * LANES, LANES)
                acc[sl] = acc[sl] + tbuf[sl]

        pltpu.sync_copy(acc, deg_hbm.at[pl.ds(cid * n_pad + base, npt)])

    return deg_kernel


@functools.lru_cache(maxsize=None)
def _make_scat_kernel(n_pad, d, per_w0, per_w1):
    npt = n_pad // NS
    wrows = 64
    nrounds = npt // wrows
    pw_max = max(per_w0, per_w1)
    nbuf = 2

    @functools.partial(
        pl.kernel,
        mesh=_sc_mesh(),
        out_type=jax.ShapeDtypeStruct((NC, n_pad, d), jnp.float32),
        scratch_types=[
            pltpu.VMEM((pw_max,), jnp.int32),
            pltpu.VMEM((pw_max,), jnp.int32),
            pltpu.VMEM((KC, d), jnp.float32),
            pltpu.VMEM((KC, d), jnp.float32),
            pltpu.VMEM_SHARED((n_pad, d), jnp.float32),
            pltpu.SemaphoreType.DMA,
            pltpu.SemaphoreType.DMA,
        ],
    )
    def scat_kernel(rows_hbm, cols_hbm, y_hbm, s_hbm, rowv, colv, gbuf0,
                    gbuf1, s_sp, sem0, sem1):
        cid = lax.axis_index("c")
        sid = lax.axis_index("s")
        gbufs = (gbuf0, gbuf1)
        sems = (sem0, sem1)
        zero16 = jnp.zeros((LANES,), jnp.float32)

        @pl.loop(0, wrows)
        def _(r):
            @pl.loop(0, d // LANES)
            def _(cc):
                gbuf0[r, pl.ds(cc * LANES, LANES)] = zero16

        @pl.loop(0, nrounds)
        def _(k):
            off = pl.multiple_of(sid * npt + k * wrows, wrows)
            pltpu.sync_copy(gbuf0.at[pl.ds(0, wrows)], s_sp.at[pl.ds(off, wrows)])

        plsc.subcore_barrier()

        def pipeline(base, per_w_c):
            nchunk = per_w_c // KC
            pltpu.sync_copy(rows_hbm.at[pl.ds(base, per_w_c)],
                            rowv.at[pl.ds(0, per_w_c)])
            pltpu.sync_copy(cols_hbm.at[pl.ds(base, per_w_c)],
                            colv.at[pl.ds(0, per_w_c)])
            for b in range(min(nbuf, nchunk)):
                eoff = pl.multiple_of(b * KC, 16)
                pltpu.async_copy(y_hbm.at[rowv.at[pl.ds(eoff, KC)]],
                                 gbufs[b], sems[b])

            @pl.loop(0, -(-nchunk // nbuf))
            def _(g):
                for b in range(nbuf):
                    ch = g * nbuf + b

                    @pl.when(ch < nchunk)
                    def _():
                        eoff = pl.multiple_of(ch * KC, 16)
                        pltpu.make_async_copy(
                            y_hbm.at[rowv.at[pl.ds(eoff, KC)]], gbufs[b],
                            sems[b]).wait()
                        pltpu.sync_copy(gbufs[b],
                                        s_sp.at[colv.at[pl.ds(eoff, KC)]],
                                        add=True)
                        nch = ch + nbuf

                        @pl.when(nch < nchunk)
                        def _():
                            noff = pl.multiple_of(nch * KC, 16)
                            pltpu.async_copy(
                                y_hbm.at[rowv.at[pl.ds(noff, KC)]],
                                gbufs[b], sems[b])

        @pl.when(cid == 0)
        def _():
            pipeline(pl.multiple_of(sid * per_w0, 16), per_w0)

        @pl.when(cid == 1)
        def _():
            pipeline(pl.multiple_of(NS * per_w0 + sid * per_w1, 16), per_w1)

        plsc.subcore_barrier()

        @pl.loop(0, nrounds)
        def _(k):
            off = pl.multiple_of(sid * npt + k * wrows, wrows)
            pltpu.sync_copy(s_sp.at[pl.ds(off, wrows)], gbuf0.at[pl.ds(0, wrows)])
            pltpu.sync_copy(gbuf0.at[pl.ds(0, wrows)],
                            s_hbm.at[cid, pl.ds(off, wrows)])

    return scat_kernel


def _lin_body(h_ref, w_ref, dis_ref, y_ref):
    x = jnp.dot(h_ref[...], w_ref[...], preferred_element_type=jnp.float32)
    y_ref[...] = x * dis_ref[...]


def _epi_body(s_ref, y_ref, dis_ref, b_ref, a_ref, lnw_ref, lnb_ref, out_ref):
    s = s_ref[0] + s_ref[1] + y_ref[...]
    pre = s * dis_ref[...] + b_ref[...]
    a = a_ref[0, 0]
    pre = jnp.where(pre >= 0, pre, a * pre)
    mean = jnp.mean(pre, axis=-1, keepdims=True)
    cent = pre - mean
    var = jnp.mean(cent * cent, axis=-1, keepdims=True)
    out_ref[...] = cent * lax.rsqrt(var + 1e-5) * lnw_ref[...] + lnb_ref[...]


def kernel(h_in, edge_index, W, b, prelu_a, ln_w, ln_b):
    n, d_in = h_in.shape
    d_out = W.shape[1]
    e = edge_index.shape[1]

    align = math.lcm(NW * LANES, NS * KC)
    ew = -(-e // align) * align
    n_pad = -(-n // (NS * WROWS)) * (NS * WROWS)

    rows = jnp.concatenate(
        [edge_index[0], jnp.zeros((ew - e,), jnp.int32)])
    cols = jnp.concatenate(
        [edge_index[1], jnp.full((ew - e,), n, jnp.int32)])

    pt = ew // NS
    per_w0 = min(max(round(F0 * pt / KC) * KC, KC), pt - KC)
    per_w1 = pt - per_w0

    deg_flat = _make_deg_kernel(n_pad, ew // NW)(cols)
    degt = deg_flat.reshape(NC, n_pad).sum(0)[:n] + 1.0
    dis = lax.rsqrt(degt)[:, None]

    br = 2000 if n % 2000 == 0 else 1000 if n % 1000 == 0 else 8
    grid = (n // br,)
    y = pl.pallas_call(
        _lin_body,
        grid=grid,
        in_specs=[
            pl.BlockSpec((br, d_in), lambda i: (i, 0)),
            pl.BlockSpec((d_in, d_out), lambda i: (0, 0)),
            pl.BlockSpec((br, 1), lambda i: (i, 0)),
        ],
        out_specs=pl.BlockSpec((br, d_out), lambda i: (i, 0)),
        out_shape=jax.ShapeDtypeStruct((n, d_out), jnp.float32),
    )(h_in, W, dis)

    s_parts = _make_scat_kernel(n_pad, d_out, per_w0, per_w1)(rows, cols, y)

    out = pl.pallas_call(
        _epi_body,
        grid=grid,
        in_specs=[
            pl.BlockSpec((NC, br, d_out), lambda i: (0, i, 0)),
            pl.BlockSpec((br, d_out), lambda i: (i, 0)),
            pl.BlockSpec((br, 1), lambda i: (i, 0)),
            pl.BlockSpec((1, d_out), lambda i: (0, 0)),
            pl.BlockSpec(memory_space=pltpu.SMEM),
            pl.BlockSpec((1, d_out), lambda i: (0, 0)),
            pl.BlockSpec((1, d_out), lambda i: (0, 0)),
        ],
        out_specs=pl.BlockSpec((br, d_out), lambda i: (i, 0)),
        out_shape=jax.ShapeDtypeStruct((n, d_out), jnp.float32),
    )(s_parts, y, dis, b.reshape(1, -1), prelu_a.reshape(1, 1),
      ln_w.reshape(1, -1), ln_b.reshape(1, -1))
    return out


# speedup vs baseline: 1.2203x; 1.1584x over previous
import functools
import math

import jax
import jax.numpy as jnp
from jax import lax
from jax.experimental import pallas as pl
from jax.experimental.pallas import tpu as pltpu
from jax.experimental.pallas import tpu_sc as plsc

NC = 2
NS = 16
LANES = 16
NW = NC * NS
KC = 80
F0 = 0.50
WROWS = 128


def _sc_mesh():
    return plsc.VectorSubcoreMesh(
        core_axis_name="c", subcore_axis_name="s", num_cores=NC, num_subcores=NS
    )


@functools.lru_cache(maxsize=None)
def _make_deg_kernel(n_pad, per_w, cbase):
    npt = n_pad // NS

    @functools.partial(
        pl.kernel,
        mesh=_sc_mesh(),
        out_type=jax.ShapeDtypeStruct((NC * n_pad,), jnp.float32),
        scratch_types=[
            pltpu.VMEM((per_w,), jnp.int32),
            pltpu.VMEM((n_pad,), jnp.float32),
            pltpu.VMEM((npt,), jnp.float32),
            pltpu.VMEM((npt,), jnp.float32),
            pltpu.VMEM_SHARED((NS * n_pad,), jnp.float32),
        ],
        compiler_params=pltpu.CompilerParams(needs_layout_passes=False),
    )
    def deg_kernel(cols_hbm, deg_hbm, colv, hist, acc, tbuf, hist_sh):
        cid = lax.axis_index("c")
        sid = lax.axis_index("s")
        wid = sid * NC + cid
        zero16 = jnp.zeros((LANES,), jnp.float32)
        one16 = jnp.ones((LANES,), jnp.float32)

        @pl.loop(0, n_pad // LANES)
        def _(r):
            hist[pl.ds(r * LANES, LANES)] = zero16

        pltpu.sync_copy(cols_hbm.at[pl.ds(cbase + wid * per_w, per_w)], colv)

        @pl.loop(0, per_w // LANES)
        def _(i):
            idx = colv[pl.ds(i * LANES, LANES)]
            plsc.addupdate_scatter(hist, [idx], one16)

        pltpu.sync_copy(hist, hist_sh.at[pl.ds(sid * n_pad, n_pad)])
        plsc.subcore_barrier()

        base = pl.multiple_of(sid * npt, npt)
        pltpu.sync_copy(hist_sh.at[pl.ds(base, npt)], acc)

        @pl.loop(1, NS)
        def _(t):
            pltpu.sync_copy(hist_sh.at[pl.ds(t * n_pad + base, npt)], tbuf)

            @pl.loop(0, npt // LANES)
            def _(j):
                sl = pl.ds(j * LANES, LANES)
                acc[sl] = acc[sl] + tbuf[sl]

        pltpu.sync_copy(acc, deg_hbm.at[pl.ds(cid * n_pad + base, npt)])

    return deg_kernel


@functools.lru_cache(maxsize=None)
def _make_scat_kernel(n_pad, d, per_w0, per_w1, rbase0, cbase0):
    npt = n_pad // NS
    wrows = 64
    nrounds = npt // wrows
    pw_max = max(per_w0, per_w1)
    nbuf = 3
    glead = 2
    h_max = -(-pw_max // (2 * KC)) * KC

    @functools.partial(
        pl.kernel,
        mesh=_sc_mesh(),
        out_type=jax.ShapeDtypeStruct((NC, n_pad, d), jnp.float32),
        scratch_types=[
            pltpu.VMEM((h_max,), jnp.int32),
            pltpu.VMEM((h_max,), jnp.int32),
            [pltpu.VMEM((KC, d), jnp.float32) for _ in range(nbuf)],
            pltpu.VMEM_SHARED((n_pad, d), jnp.float32),
            [pltpu.SemaphoreType.DMA for _ in range(nbuf)],
            [pltpu.SemaphoreType.DMA for _ in range(nbuf)],
        ],
    )
    def scat_kernel(edges_hbm, y_hbm, s_hbm, rowv, colv, gbufs, s_sp,
                    gsems, ssems):
        cid = lax.axis_index("c")
        sid = lax.axis_index("s")
        zero16 = jnp.zeros((LANES,), jnp.float32)

        @pl.loop(0, wrows)
        def _(r):
            @pl.loop(0, d // LANES)
            def _(cc):
                gbufs[0][r, pl.ds(cc * LANES, LANES)] = zero16

        @pl.loop(0, nrounds)
        def _(k):
            off = pl.multiple_of(sid * npt + k * wrows, wrows)
            pltpu.sync_copy(gbufs[0].at[pl.ds(0, wrows)],
                            s_sp.at[pl.ds(off, wrows)])

        plsc.subcore_barrier()

        def segment(rbase, cbase, ssize):
            nchunk = ssize // KC
            pltpu.sync_copy(edges_hbm.at[pl.ds(rbase, ssize)],
                            rowv.at[pl.ds(0, ssize)])
            pltpu.sync_copy(edges_hbm.at[pl.ds(cbase, ssize)],
                            colv.at[pl.ds(0, ssize)])

            def gather(ch, b):
                eoff = pl.multiple_of(ch * KC, 16)
                pltpu.async_copy(y_hbm.at[rowv.at[pl.ds(eoff, KC)]],
                                 gbufs[b], gsems[b])

            def scat_desc(ch, b):
                eoff = pl.multiple_of(ch * KC, 16)
                return pltpu.make_async_copy(
                    gbufs[b], s_sp.at[colv.at[pl.ds(eoff, KC)]], ssems[b])

            for b in range(min(glead, nchunk)):
                gather(b, b)

            @pl.loop(0, -(-nchunk // nbuf))
            def _(g):
                for b in range(nbuf):
                    ch = g * nbuf + b

                    @pl.when(ch < nchunk)
                    def _():
                        eoff = pl.multiple_of(ch * KC, 16)
                        pltpu.make_async_copy(
                            y_hbm.at[rowv.at[pl.ds(eoff, KC)]], gbufs[b],
                            gsems[b]).wait()
                        pltpu.async_copy(
                            gbufs[b], s_sp.at[colv.at[pl.ds(eoff, KC)]],
                            ssems[b], add=True)
                        nch = ch + glead
                        nb = (b + glead) % nbuf

                        @pl.when(nch < nchunk)
                        def _():
                            @pl.when(nch - nbuf >= 0)
                            def _():
                                scat_desc(nch - nbuf, nb).wait()
                            gather(nch, nb)

            for b in range(min(nbuf, nchunk)):
                last = nchunk - 1 - ((nchunk - 1 - b) % nbuf)
                scat_desc(last, b).wait()

        def pipeline(rbase, cbase, per_w_c):
            h0 = -(-per_w_c // (2 * KC)) * KC
            segment(rbase, cbase, h0)
            if per_w_c - h0 > 0:
                segment(rbase + h0, cbase + h0, per_w_c - h0)

        @pl.when(cid == 0)
        def _():
            pipeline(pl.multiple_of(rbase0 + sid * per_w0, 16),
                     pl.multiple_of(cbase0 + sid * per_w0, 16), per_w0)

        @pl.when(cid == 1)
        def _():
            pipeline(
                pl.multiple_of(rbase0 + NS * per_w0 + sid * per_w1, 16),
                pl.multiple_of(cbase0 + NS * per_w0 + sid * per_w1, 16),
                per_w1)

        plsc.subcore_barrier()

        @pl.loop(0, nrounds)
        def _(k):
            off = pl.multiple_of(sid * npt + k * wrows, wrows)
            pltpu.sync_copy(s_sp.at[pl.ds(off, wrows)], gbufs[0].at[pl.ds(0, wrows)])
            pltpu.sync_copy(gbufs[0].at[pl.ds(0, wrows)],
                            s_hbm.at[cid, pl.ds(off, wrows)])

    return scat_kernel


def _lin_body(h_ref, w_ref, dis_ref, y_ref):
    x = jnp.dot(h_ref[...], w_ref[...], preferred_element_type=jnp.float32)
    y_ref[...] = x * dis_ref[...]


def _epi_body(s_ref, y_ref, dis_ref, b_ref, a_ref, lnw_ref, lnb_ref, out_ref):
    s = s_ref[0] + s_ref[1] + y_ref[...]
    pre = s * dis_ref[...] + b_ref[...]
    a = a_ref[0, 0]
    pre = jnp.where(pre >= 0, pre, a * pre)
    mean = jnp.mean(pre, axis=-1, keepdims=True)
    cent = pre - mean
    var = jnp.mean(cent * cent, axis=-1, keepdims=True)
    out_ref[...] = cent * lax.rsqrt(var + 1e-5) * lnw_ref[...] + lnb_ref[...]


def kernel(h_in, edge_index, W, b, prelu_a, ln_w, ln_b):
    n, d_in = h_in.shape
    d_out = W.shape[1]
    e = edge_index.shape[1]

    align = math.lcm(NW * LANES, NS * KC)
    ew = -(-e // align) * align
    n_pad = -(-n // (NS * WROWS)) * (NS * WROWS)

    if ew == e:
        edges = edge_index.reshape(-1)
    else:
        rows = jnp.concatenate(
            [edge_index[0], jnp.zeros((ew - e,), jnp.int32)])
        cols = jnp.concatenate(
            [edge_index[1], jnp.full((ew - e,), n, jnp.int32)])
        edges = jnp.concatenate([rows, cols])
    cbase = ew

    pt = ew // NS
    per_w0 = min(max(round(F0 * pt / KC) * KC, KC), pt - KC)
    per_w1 = pt - per_w0

    deg_flat = _make_deg_kernel(n_pad, ew // NW, cbase)(edges)
    degt = deg_flat.reshape(NC, n_pad).sum(0)[:n] + 1.0
    dis = lax.rsqrt(degt)[:, None]

    br = 2000 if n % 2000 == 0 else 1000 if n % 1000 == 0 else 8
    grid = (n // br,)
    y = pl.pallas_call(
        _lin_body,
        grid=grid,
        in_specs=[
            pl.BlockSpec((br, d_in), lambda i: (i, 0)),
            pl.BlockSpec((d_in, d_out), lambda i: (0, 0)),
            pl.BlockSpec((br, 1), lambda i: (i, 0)),
        ],
        out_specs=pl.BlockSpec((br, d_out), lambda i: (i, 0)),
        out_shape=jax.ShapeDtypeStruct((n, d_out), jnp.float32),
    )(h_in, W, dis)

    s_parts = _make_scat_kernel(n_pad, d_out, per_w0, per_w1, 0, cbase)(
        edges, y)

    out = pl.pallas_call(
        _epi_body,
        grid=grid,
        in_specs=[
            pl.BlockSpec((NC, br, d_out), lambda i: (0, i, 0)),
            pl.BlockSpec((br, d_out), lambda i: (i, 0)),
            pl.BlockSpec((br, 1), lambda i: (i, 0)),
            pl.BlockSpec((1, d_out), lambda i: (0, 0)),
            pl.BlockSpec(memory_space=pltpu.SMEM),
            pl.BlockSpec((1, d_out), lambda i: (0, 0)),
            pl.BlockSpec((1, d_out), lambda i: (0, 0)),
        ],
        out_specs=pl.BlockSpec((br, d_out), lambda i: (i, 0)),
        out_shape=jax.ShapeDtypeStruct((n, d_out), jnp.float32),
    )(s_parts, y, dis, b.reshape(1, -1), prelu_a.reshape(1, 1),
      ln_w.reshape(1, -1), ln_b.reshape(1, -1))
    return out


# speedup vs baseline: 1.2437x; 1.0192x over previous
import functools
import math

import jax
import jax.numpy as jnp
from jax import lax
from jax.experimental import pallas as pl
from jax.experimental.pallas import tpu as pltpu
from jax.experimental.pallas import tpu_sc as plsc

NC = 2
NS = 16
LANES = 16
NW = NC * NS
KC = 80
F0 = 0.50
WROWS = 128


def _sc_mesh():
    return plsc.VectorSubcoreMesh(
        core_axis_name="c", subcore_axis_name="s", num_cores=NC, num_subcores=NS
    )


@functools.lru_cache(maxsize=None)
def _make_deg_kernel(n_pad, per_w, cbase):
    npt = n_pad // NS

    @functools.partial(
        pl.kernel,
        mesh=_sc_mesh(),
        out_type=jax.ShapeDtypeStruct((NC * n_pad,), jnp.float32),
        scratch_types=[
            pltpu.VMEM((per_w,), jnp.int32),
            pltpu.VMEM((n_pad,), jnp.float32),
            pltpu.VMEM((npt,), jnp.float32),
            pltpu.VMEM((npt,), jnp.float32),
            pltpu.VMEM_SHARED((NS * n_pad,), jnp.float32),
        ],
        compiler_params=pltpu.CompilerParams(needs_layout_passes=False),
    )
    def deg_kernel(cols_hbm, deg_hbm, colv, hist, acc, tbuf, hist_sh):
        cid = lax.axis_index("c")
        sid = lax.axis_index("s")
        wid = sid * NC + cid
        zero16 = jnp.zeros((LANES,), jnp.float32)
        one16 = jnp.ones((LANES,), jnp.float32)

        @pl.loop(0, n_pad // LANES)
        def _(r):
            hist[pl.ds(r * LANES, LANES)] = zero16

        pltpu.sync_copy(cols_hbm.at[pl.ds(cbase + wid * per_w, per_w)], colv)

        @pl.loop(0, per_w // LANES, unroll=4)
        def _(i):
            idx = colv[pl.ds(i * LANES, LANES)]
            plsc.addupdate_scatter(hist, [idx], one16)

        pltpu.sync_copy(hist, hist_sh.at[pl.ds(sid * n_pad, n_pad)])
        plsc.subcore_barrier()

        base = pl.multiple_of(sid * npt, npt)
        pltpu.sync_copy(hist_sh.at[pl.ds(base, npt)], acc)

        @pl.loop(1, NS)
        def _(t):
            pltpu.sync_copy(hist_sh.at[pl.ds(t * n_pad + base, npt)], tbuf)

            @pl.loop(0, npt // LANES)
            def _(j):
                sl = pl.ds(j * LANES, LANES)
                acc[sl] = acc[sl] + tbuf[sl]

        pltpu.sync_copy(acc, deg_hbm.at[pl.ds(cid * n_pad + base, npt)])

    return deg_kernel


@functools.lru_cache(maxsize=None)
def _make_scat_kernel(n_pad, d, per_w0, per_w1, rbase0, cbase0):
    npt = n_pad // NS
    wrows = KC
    nrounds = npt // wrows
    pw_max = max(per_w0, per_w1)
    nbuf = 3
    glead = 2
    h_max = -(-pw_max // (2 * KC)) * KC

    @functools.partial(
        pl.kernel,
        mesh=_sc_mesh(),
        out_type=jax.ShapeDtypeStruct((NC, n_pad, d), jnp.float32),
        scratch_types=[
            pltpu.VMEM((h_max,), jnp.int32),
            pltpu.VMEM((h_max,), jnp.int32),
            [pltpu.VMEM((KC, d), jnp.float32) for _ in range(nbuf)],
            pltpu.VMEM_SHARED((n_pad, d), jnp.float32),
            [pltpu.SemaphoreType.DMA for _ in range(nbuf)],
            [pltpu.SemaphoreType.DMA for _ in range(nbuf)],
        ],
    )
    def scat_kernel(edges_hbm, y_hbm, s_hbm, rowv, colv, gbufs, s_sp,
                    gsems, ssems):
        cid = lax.axis_index("c")
        sid = lax.axis_index("s")
        zero16 = jnp.zeros((LANES,), jnp.float32)

        @pl.loop(0, wrows)
        def _(r):
            @pl.loop(0, d // LANES)
            def _(cc):
                gbufs[0][r, pl.ds(cc * LANES, LANES)] = zero16

        for k in range(nrounds):
            off = pl.multiple_of(sid * npt + k * wrows, wrows)
            pltpu.async_copy(gbufs[0], s_sp.at[pl.ds(off, wrows)], gsems[0])
        for k in range(nrounds):
            off = pl.multiple_of(sid * npt + k * wrows, wrows)
            pltpu.make_async_copy(gbufs[0], s_sp.at[pl.ds(off, wrows)],
                                  gsems[0]).wait()

        plsc.subcore_barrier()

        def segment(rbase, cbase, ssize):
            nchunk = ssize // KC
            pltpu.sync_copy(edges_hbm.at[pl.ds(rbase, ssize)],
                            rowv.at[pl.ds(0, ssize)])
            pltpu.sync_copy(edges_hbm.at[pl.ds(cbase, ssize)],
                            colv.at[pl.ds(0, ssize)])

            def gather(ch, b):
                eoff = pl.multiple_of(ch * KC, 16)
                pltpu.async_copy(y_hbm.at[rowv.at[pl.ds(eoff, KC)]],
                                 gbufs[b], gsems[b])

            def scat_desc(ch, b):
                eoff = pl.multiple_of(ch * KC, 16)
                return pltpu.make_async_copy(
                    gbufs[b], s_sp.at[colv.at[pl.ds(eoff, KC)]], ssems[b])

            for b in range(min(glead, nchunk)):
                gather(b, b)

            @pl.loop(0, -(-nchunk // nbuf))
            def _(g):
                for b in range(nbuf):
                    ch = g * nbuf + b

                    @pl.when(ch < nchunk)
                    def _():
                        eoff = pl.multiple_of(ch * KC, 16)
                        pltpu.make_async_copy(
                            y_hbm.at[rowv.at[pl.ds(eoff, KC)]], gbufs[b],
                            gsems[b]).wait()
                        pltpu.async_copy(
                            gbufs[b], s_sp.at[colv.at[pl.ds(eoff, KC)]],
                            ssems[b], add=True)
                        nch = ch + glead
                        nb = (b + glead) % nbuf

                        @pl.when(nch < nchunk)
                        def _():
                            @pl.when(nch - nbuf >= 0)
                            def _():
                                scat_desc(nch - nbuf, nb).wait()
                            gather(nch, nb)

            for b in range(min(nbuf, nchunk)):
                last = nchunk - 1 - ((nchunk - 1 - b) % nbuf)
                scat_desc(last, b).wait()

        def pipeline(rbase, cbase, per_w_c):
            h0 = -(-per_w_c // (2 * KC)) * KC
            segment(rbase, cbase, h0)
            if per_w_c - h0 > 0:
                segment(rbase + h0, cbase + h0, per_w_c - h0)

        @pl.when(cid == 0)
        def _():
            pipeline(pl.multiple_of(rbase0 + sid * per_w0, 16),
                     pl.multiple_of(cbase0 + sid * per_w0, 16), per_w0)

        @pl.when(cid == 1)
        def _():
            pipeline(
                pl.multiple_of(rbase0 + NS * per_w0 + sid * per_w1, 16),
                pl.multiple_of(cbase0 + NS * per_w0 + sid * per_w1, 16),
                per_w1)

        plsc.subcore_barrier()

        def wout_desc(k, bb):
            off = pl.multiple_of(sid * npt + k * wrows, wrows)
            return pltpu.make_async_copy(gbufs[bb],
                                         s_hbm.at[cid, pl.ds(off, wrows)],
                                         ssems[bb])

        for k in range(nrounds):
            bb = k % 2
            if k >= 2:
                wout_desc(k - 2, bb).wait()
            off = pl.multiple_of(sid * npt + k * wrows, wrows)
            pltpu.sync_copy(s_sp.at[pl.ds(off, wrows)], gbufs[bb])
            pltpu.async_copy(gbufs[bb], s_hbm.at[cid, pl.ds(off, wrows)],
                             ssems[bb])
        for k in range(max(nrounds - 2, 0), nrounds):
            wout_desc(k, k % 2).wait()

    return scat_kernel


def _lin_body(h_ref, w_ref, dis_ref, y_ref):
    x = jnp.dot(h_ref[...], w_ref[...], preferred_element_type=jnp.float32)
    y_ref[...] = x * dis_ref[...]


def _epi_body(s_ref, y_ref, dis_ref, b_ref, a_ref, lnw_ref, lnb_ref, out_ref):
    s = s_ref[0] + s_ref[1] + y_ref[...]
    pre = s * dis_ref[...] + b_ref[...]
    a = a_ref[0, 0]
    pre = jnp.where(pre >= 0, pre, a * pre)
    mean = jnp.mean(pre, axis=-1, keepdims=True)
    cent = pre - mean
    var = jnp.mean(cent * cent, axis=-1, keepdims=True)
    out_ref[...] = cent * lax.rsqrt(var + 1e-5) * lnw_ref[...] + lnb_ref[...]


def kernel(h_in, edge_index, W, b, prelu_a, ln_w, ln_b):
    n, d_in = h_in.shape
    d_out = W.shape[1]
    e = edge_index.shape[1]

    align = math.lcm(NW * LANES, NS * KC)
    ew = -(-e // align) * align
    n_pad = -(-n // (NS * WROWS)) * (NS * WROWS)

    if ew == e:
        edges = edge_index.reshape(-1)
    else:
        rows = jnp.concatenate(
            [edge_index[0], jnp.zeros((ew - e,), jnp.int32)])
        cols = jnp.concatenate(
            [edge_index[1], jnp.full((ew - e,), n, jnp.int32)])
        edges = jnp.concatenate([rows, cols])
    cbase = ew

    pt = ew // NS
    per_w0 = min(max(round(F0 * pt / KC) * KC, KC), pt - KC)
    per_w1 = pt - per_w0

    deg_flat = _make_deg_kernel(n_pad, ew // NW, cbase)(edges)
    degt = deg_flat.reshape(NC, n_pad).sum(0)[:n] + 1.0
    dis = lax.rsqrt(degt)[:, None]

    br = 2000 if n % 2000 == 0 else 1000 if n % 1000 == 0 else 8
    grid = (n // br,)
    y = pl.pallas_call(
        _lin_body,
        grid=grid,
        in_specs=[
            pl.BlockSpec((br, d_in), lambda i: (i, 0)),
            pl.BlockSpec((d_in, d_out), lambda i: (0, 0)),
            pl.BlockSpec((br, 1), lambda i: (i, 0)),
        ],
        out_specs=pl.BlockSpec((br, d_out), lambda i: (i, 0)),
        out_shape=jax.ShapeDtypeStruct((n, d_out), jnp.float32),
    )(h_in, W, dis)

    s_parts = _make_scat_kernel(n_pad, d_out, per_w0, per_w1, 0, cbase)(
        edges, y)

    out = pl.pallas_call(
        _epi_body,
        grid=grid,
        in_specs=[
            pl.BlockSpec((NC, br, d_out), lambda i: (0, i, 0)),
            pl.BlockSpec((br, d_out), lambda i: (i, 0)),
            pl.BlockSpec((br, 1), lambda i: (i, 0)),
            pl.BlockSpec((1, d_out), lambda i: (0, 0)),
            pl.BlockSpec(memory_space=pltpu.SMEM),
            pl.BlockSpec((1, d_out), lambda i: (0, 0)),
            pl.BlockSpec((1, d_out), lambda i: (0, 0)),
        ],
        out_specs=pl.BlockSpec((br, d_out), lambda i: (i, 0)),
        out_shape=jax.ShapeDtypeStruct((n, d_out), jnp.float32),
    )(s_parts, y, dis, b.reshape(1, -1), prelu_a.reshape(1, 1),
      ln_w.reshape(1, -1), ln_b.reshape(1, -1))
    return out
